# Initial kernel scaffold; baseline (speedup 1.0000x reference)
import jax
import jax.numpy as jnp
from jax.experimental import pallas as pl


def kernel(coords, type_params, global_params, block_types, min_block_bondsep, bt_atom_types, bt_path_distance):
    raise NotImplementedError("write your pallas kernel here")


# speedup vs baseline: 838.1211x; 838.1211x over previous
import jax
import jax.numpy as jnp
import numpy as np
from jax.experimental import pallas as pl

_P = 2
_B = 64
_A = 32
_N = _B * _A
_TI = 256
_NT = _N // _TI
_TB = _TI // _A
_C_LK = 1.0 / (2.0 * np.pi ** 1.5)


def _pair_tile_kernel(rp_ref, cp_ref, w_ref, dl_ref, out_ref):
    t = pl.program_id(1)
    rp = rp_ref[0]
    cp = cp_ref[0]
    wt = w_ref[0]
    dl = dl_ref[0]

    cross = jnp.dot(rp[:, 0:8], cp[0:8, :], preferred_element_type=jnp.float32,
                    precision=jax.lax.Precision.HIGHEST)
    n2i = rp[:, 8:9]
    n2j = cp[8:9, :]
    d2 = jnp.maximum(n2i + n2j - 2.0 * cross, 0.0)
    d = jnp.sqrt(d2 + 1e-8)
    d = jnp.maximum(d, 0.8)
    inv_d = 1.0 / d
    inv_d2 = inv_d * inv_d

    ri = rp[:, 9:10]
    rj = cp[9:10, :]
    sd = (ri + rj) * inv_d
    sd2 = sd * sd
    sd6 = sd2 * sd2 * sd2
    epsp = jnp.sqrt(rp[:, 10:11] * cp[10:11, :] + 1e-12)
    lj = epsp * (sd6 * sd6 - 2.0 * sd6)

    xi = (d - ri) * rp[:, 11:12]
    xj = (d - rj) * cp[11:12, :]
    ei = jnp.exp(-xi * xi)
    ej = jnp.exp(-xj * xj)
    lk = _C_LK * inv_d2 * (rp[:, 12:13] * cp[13:14, :] * ei
                           + cp[12:13, :] * rp[:, 13:14] * ej)
    hp = rp[:, 14:15] * cp[14:15, :]

    rowi = jax.lax.broadcasted_iota(jnp.int32, (_TI, 1), 0)
    colj = jax.lax.broadcasted_iota(jnp.int32, (1, _N), 1)
    same = (rowi // _A + t * _TB) == (colj // _A)
    er_i = jax.lax.broadcasted_iota(jnp.int32, (_TI, _TB), 0) // _A
    er_j = jax.lax.broadcasted_iota(jnp.int32, (_TI, _TB), 1)
    erow = (er_i == er_j).astype(jnp.float32)
    w_exp = jnp.dot(erow, wt, preferred_element_type=jnp.float32)
    tm_i = jax.lax.broadcasted_iota(jnp.int32, (_A, _N), 0)
    tm_j = jax.lax.broadcasted_iota(jnp.int32, (_A, _N), 1) % _A
    tmat = (tm_i == tm_j).astype(jnp.float32)
    d_exp = jnp.dot(dl, tmat, preferred_element_type=jnp.float32)
    w = w_exp + jnp.where(same, d_exp, 0.0)

    cut = (d < 6.0).astype(jnp.float32)
    wm = w * cut
    lj_s = jnp.sum(lj * wm)
    lk_s = jnp.sum(lk * wm * hp)

    ii = jax.lax.broadcasted_iota(jnp.int32, (8, 128), 0)
    upd = jnp.where(ii == 0, lj_s, 0.0) + jnp.where(ii == 1, lk_s, 0.0)

    @pl.when(t == 0)
    def _init():
        out_ref[0] = upd

    @pl.when(t != 0)
    def _acc():
        out_ref[0] = out_ref[0] + upd


def _pairwise_call(rp, cp, w_colexp, delta_rows, interpret=False):
    return pl.pallas_call(
        _pair_tile_kernel,
        grid=(_P, _NT),
        in_specs=[
            pl.BlockSpec((1, _TI, 16), lambda p, t: (p, t, 0)),
            pl.BlockSpec((1, 16, _N), lambda p, t: (p, 0, 0)),
            pl.BlockSpec((1, _TB, _N), lambda p, t: (p, t, 0)),
            pl.BlockSpec((1, _TI, _A), lambda p, t: (p, t, 0)),
        ],
        out_specs=pl.BlockSpec((1, 8, 128), lambda p, t: (p, 0, 0)),
        out_shape=jax.ShapeDtypeStruct((_P, 8, 128), jnp.float32),
        interpret=interpret,
    )(rp, cp, w_colexp, delta_rows)


def kernel(coords, type_params, global_params, block_types, min_block_bondsep,
           bt_atom_types, bt_path_distance):
    at = bt_atom_types[block_types].reshape(_P, _N)
    pa = type_params[at]
    r = pa[..., 0]
    wd = pa[..., 1]
    invlam = 1.0 / pa[..., 3]
    a = pa[..., 2] * invlam
    vol = pa[..., 4]
    h = (pa[..., 7] < 0.5).astype(jnp.float32)

    x = coords[..., 0]
    y = coords[..., 1]
    z = coords[..., 2]
    n2 = x * x + y * y + z * z
    zeros = jnp.zeros((_P, _N), jnp.float32)
    feats = jnp.stack([x, y, z, zeros, zeros, zeros, zeros, zeros,
                       n2, r, wd, invlam, a, vol, h, zeros], axis=-1)
    cp = jnp.transpose(feats, (0, 2, 1))

    w_inter = jnp.where(min_block_bondsep > 4, 1.0,
                        jnp.where(min_block_bondsep == 4, 0.2, 0.0)).astype(jnp.float32)
    w_colexp = jnp.repeat(w_inter, _A, axis=2)

    pd_intra = bt_path_distance[block_types]
    w_intra = jnp.where(pd_intra > 4, 1.0,
                        jnp.where(pd_intra == 4, 0.2, 0.0)).astype(jnp.float32)
    eye = jnp.eye(_A, dtype=bool)
    w_true = jnp.where(eye[None, None], 0.0, w_intra)
    w_diag = jnp.diagonal(w_inter, axis1=1, axis2=2)
    delta_rows = (w_true - w_diag[:, :, None, None]).reshape(_P, _N, _A)

    out = _pairwise_call(feats, cp, w_colexp, delta_rows)
    return jnp.stack([0.5 * out[:, 0, 0], 0.5 * out[:, 1, 0]], axis=0)


# speedup vs baseline: 954.2764x; 1.1386x over previous
import jax
import jax.numpy as jnp
import numpy as np
from jax.experimental import pallas as pl

_P = 2
_B = 64
_A = 32
_N = _B * _A
_TI = 256
_NT = _N // _TI
_TB = _TI // _A
_C_LK = 1.0 / (2.0 * np.pi ** 1.5)


def _pair_tile_kernel(rp_ref, cp_ref, w_ref, dl_ref, out_ref):
    t = pl.program_id(1)
    rp = rp_ref[0]
    cp = cp_ref[0]
    wt = w_ref[0]
    dl = dl_ref[0]

    cross = jnp.dot(rp[:, 0:8], cp[0:8, :], preferred_element_type=jnp.float32,
                    precision=jax.lax.Precision.HIGHEST)
    n2i = rp[:, 8:9]
    n2j = cp[8:9, :]
    t2 = jnp.maximum(n2i + n2j - 2.0 * cross, 0.0) + 1e-8
    rs = jax.lax.rsqrt(t2)
    d = jnp.maximum(t2 * rs, 0.8)
    inv_d = jnp.minimum(rs, 1.25)
    inv_d2 = inv_d * inv_d

    ri = rp[:, 9:10]
    rj = cp[9:10, :]
    sd = (ri + rj) * inv_d
    sd2 = sd * sd
    sd6 = sd2 * sd2 * sd2
    lj = (rp[:, 10:11] * cp[10:11, :]) * (sd6 * sd6 - 2.0 * sd6)

    xi = (d - ri) * rp[:, 11:12]
    xj = (d - rj) * cp[11:12, :]
    ei = jnp.exp(-xi * xi)
    ej = jnp.exp(-xj * xj)
    lk = inv_d2 * (rp[:, 12:13] * cp[13:14, :] * ei
                   + cp[12:13, :] * rp[:, 13:14] * ej)

    rowi = jax.lax.broadcasted_iota(jnp.int32, (_TI, 1), 0)
    colj = jax.lax.broadcasted_iota(jnp.int32, (1, _N), 1)
    same = (rowi // _A + t * _TB) == (colj // _A)
    er_i = jax.lax.broadcasted_iota(jnp.int32, (_TI, _TB), 0) // _A
    er_j = jax.lax.broadcasted_iota(jnp.int32, (_TI, _TB), 1)
    erow = (er_i == er_j).astype(jnp.float32)
    w_exp = jnp.dot(erow, wt, preferred_element_type=jnp.float32)
    tm_i = jax.lax.broadcasted_iota(jnp.int32, (_A, _N), 0)
    tm_j = jax.lax.broadcasted_iota(jnp.int32, (_A, _N), 1) % _A
    tmat = (tm_i == tm_j).astype(jnp.float32)
    d_exp = jnp.dot(dl, tmat, preferred_element_type=jnp.float32)
    w = w_exp + jnp.where(same, d_exp, 0.0)

    wm = jnp.where(d < 6.0, w, 0.0)
    lj_s = jnp.sum(lj * wm)
    lk_s = jnp.sum(lk * wm)

    ii = jax.lax.broadcasted_iota(jnp.int32, (8, 128), 0)
    upd = jnp.where(ii == 0, lj_s, 0.0) + jnp.where(ii == 1, lk_s, 0.0)

    @pl.when(t == 0)
    def _init():
        out_ref[0] = upd

    @pl.when(t != 0)
    def _acc():
        out_ref[0] = out_ref[0] + upd


def _pairwise_call(rp, cp, w_colexp, delta_rows, interpret=False):
    return pl.pallas_call(
        _pair_tile_kernel,
        grid=(_P, _NT),
        in_specs=[
            pl.BlockSpec((1, _TI, 16), lambda p, t: (p, t, 0)),
            pl.BlockSpec((1, 16, _N), lambda p, t: (p, 0, 0)),
            pl.BlockSpec((1, _TB, _N), lambda p, t: (p, t, 0)),
            pl.BlockSpec((1, _TI, _A), lambda p, t: (p, t, 0)),
        ],
        out_specs=pl.BlockSpec((1, 8, 128), lambda p, t: (p, 0, 0)),
        out_shape=jax.ShapeDtypeStruct((_P, 8, 128), jnp.float32),
        interpret=interpret,
    )(rp, cp, w_colexp, delta_rows)


def kernel(coords, type_params, global_params, block_types, min_block_bondsep,
           bt_atom_types, bt_path_distance):
    at = bt_atom_types[block_types].reshape(_P, _N)
    pa = type_params[at]
    r = pa[..., 0]
    sw = jnp.sqrt(pa[..., 1])
    invlam = 1.0 / pa[..., 3]
    h = (pa[..., 7] < 0.5).astype(jnp.float32)
    a = _C_LK * h * pa[..., 2] * invlam
    vol = h * pa[..., 4]

    x = coords[..., 0]
    y = coords[..., 1]
    z = coords[..., 2]
    n2 = x * x + y * y + z * z
    zeros = jnp.zeros((_P, _N), jnp.float32)
    feats = jnp.stack([x, y, z, zeros, zeros, zeros, zeros, zeros,
                       n2, r, sw, invlam, a, vol, zeros, zeros], axis=-1)
    cp = jnp.transpose(feats, (0, 2, 1))

    w_inter = jnp.where(min_block_bondsep > 4, 1.0,
                        jnp.where(min_block_bondsep == 4, 0.2, 0.0)).astype(jnp.float32)
    w_colexp = jnp.repeat(w_inter, _A, axis=2)

    pd_intra = bt_path_distance[block_types]
    w_intra = jnp.where(pd_intra > 4, 1.0,
                        jnp.where(pd_intra == 4, 0.2, 0.0)).astype(jnp.float32)
    eye = jnp.eye(_A, dtype=bool)
    w_true = jnp.where(eye[None, None], 0.0, w_intra)
    w_diag = jnp.diagonal(w_inter, axis1=1, axis2=2)
    delta_rows = (w_true - w_diag[:, :, None, None]).reshape(_P, _N, _A)

    out = _pairwise_call(feats, cp, w_colexp, delta_rows)
    return jnp.stack([0.5 * out[:, 0, 0], 0.5 * out[:, 1, 0]], axis=0)
